# baseline (speedup 1.0000x reference)
import functools

import jax
import jax.numpy as jnp
from jax import lax
from jax.experimental import pallas as pl
from jax.experimental.pallas import tpu as pltpu
from jax.experimental.pallas import tpu_sc as plsc

NUM_NODES = 50000
VOCAB_SIZE = 100000
EMBEDDING_DIM = 64
NNZ = 800000

NUM_SC = 2
NUM_TILES = 16
HALF = NUM_NODES // NUM_SC
ACC_ROWS = 25088
DUMMY = HALF
CHUNK = 128
CPT = 392
NCHUNKS = CPT * NUM_TILES
NNZ_PAD = NCHUNKS * CHUNK
STRIP = 4
NSTRIP = CPT // STRIP
RING = 3
MROWS = 3 * STRIP
ZROWS = ACC_ROWS // NUM_TILES
WROWS = 1560

_mesh = plsc.VectorSubcoreMesh(core_axis_name="c", subcore_axis_name="s")


@functools.partial(
    pl.kernel,
    out_type=jax.ShapeDtypeStruct((NUM_NODES, EMBEDDING_DIM), jnp.float32),
    mesh=_mesh,
    scratch_types=[
        pltpu.VMEM_SHARED((ACC_ROWS, EMBEDDING_DIM), jnp.float32),
        pltpu.VMEM((MROWS, CHUNK), jnp.int32),
        pltpu.VMEM((MROWS, CHUNK), jnp.int32),
        pltpu.VMEM((MROWS, CHUNK), jnp.float32),
        pltpu.VMEM((RING, CHUNK, EMBEDDING_DIM), jnp.float32),
        pltpu.SemaphoreType.DMA,
        pltpu.SemaphoreType.DMA,
        pltpu.SemaphoreType.DMA,
    ],
    compiler_params=pltpu.CompilerParams(use_tc_tiling_on_sc=False),
)
def _sc_aggregate(table, srcx, idxx, valx, zeros, out,
                  acc, src_b, idx_b, val_b, rows_b, msem, gsem, ssem):
    c = lax.axis_index("c")
    s = lax.axis_index("s")
    node_base = c * HALF

    pltpu.sync_copy(zeros, acc.at[pl.ds(s * ZROWS, ZROWS)])
    plsc.subcore_barrier()

    chunk0 = s * CPT

    def load_strip(st, ring_slot, mode):
        g0 = chunk0 + st * STRIP
        r0 = ring_slot * STRIP
        pairs = [
            (srcx.at[pl.ds(g0, STRIP)], src_b.at[pl.ds(r0, STRIP)]),
            (idxx.at[c, pl.ds(g0, STRIP)], idx_b.at[pl.ds(r0, STRIP)]),
            (valx.at[pl.ds(g0, STRIP)], val_b.at[pl.ds(r0, STRIP)]),
        ]
        for a, b in pairs:
            if mode == "sync":
                pltpu.sync_copy(a, b)
            elif mode == "async":
                pltpu.async_copy(a, b, msem)
            else:
                pltpu.make_async_copy(a, b, msem).wait()

    def fire_gather(t):
        mrow = t % MROWS
        pltpu.async_copy(table.at[src_b.at[mrow]], rows_b.at[t % RING], gsem)

    load_strip(0, 0, "sync")
    load_strip(1, 1, "sync")
    load_strip(2, 2, "async")
    for t in range(2):
        fire_gather(t)

    @pl.loop(0, CPT)
    def step(t):
        @pl.when(t >= 1)
        def _():
            pltpu.make_async_copy(table.at[pl.ds(0, CHUNK)],
                                  rows_b.at[(t - 1) % RING], ssem).wait()

        st = t // STRIP

        @pl.when((t > 0) & (t % STRIP == 0))
        def _():
            @pl.when(st + 1 <= NSTRIP - 1)
            def _():
                load_strip(st + 1, (st + 1) % 3, "wait")

            @pl.when(st + 2 <= NSTRIP - 1)
            def _():
                load_strip(st + 2, (st + 2) % 3, "async")

        @pl.when(t + 2 <= CPT - 1)
        def _():
            fire_gather(t + 2)

        b = t % RING
        mrow = t % MROWS
        pltpu.make_async_copy(table.at[src_b.at[mrow]], rows_b.at[b],
                              gsem).wait()

        @pl.loop(0, CHUNK // 16)
        def scale(i):
            vv = val_b[mrow, pl.ds(i * 16, 16)]
            for k in range(16):
                e = i * 16 + k
                sv = vv[k]
                for q in range(EMBEDDING_DIM // 16):
                    rows_b[b, e, pl.ds(q * 16, 16)] = (
                        rows_b[b, e, pl.ds(q * 16, 16)] * sv)

        pltpu.async_copy(rows_b.at[b], acc.at[idx_b.at[mrow]], ssem, add=True)

    for tt in range(CPT - 1, CPT):
        pltpu.make_async_copy(table.at[pl.ds(0, CHUNK)],
                              rows_b.at[tt % RING], ssem).wait()
    plsc.subcore_barrier()

    pltpu.sync_copy(acc.at[pl.ds(s * WROWS, WROWS)],
                    out.at[pl.ds(node_base + s * WROWS, WROWS)])
    rem = HALF - NUM_TILES * WROWS

    @pl.when(s == 0)
    def _():
        pltpu.sync_copy(acc.at[pl.ds(NUM_TILES * WROWS, rem)],
                        out.at[pl.ds(node_base + NUM_TILES * WROWS, rem)])


def kernel(token_embeddings, operator_values, operator_indices):
    pad = NNZ_PAD - NNZ
    dst = jnp.concatenate(
        [operator_indices[:, 0], jnp.full((pad,), NUM_NODES, jnp.int32)])
    src = jnp.concatenate(
        [operator_indices[:, 1], jnp.zeros((pad,), jnp.int32)])
    vals = jnp.concatenate(
        [operator_values, jnp.zeros((pad,), jnp.float32)])
    halves = jnp.arange(NUM_SC, dtype=jnp.int32)[:, None] * HALF
    local = dst[None, :] - halves
    idxx = jnp.where((local >= 0) & (local < HALF), local, DUMMY)
    zeros = jnp.zeros((ZROWS, EMBEDDING_DIM), jnp.float32)
    return _sc_aggregate(
        token_embeddings,
        src.reshape(NCHUNKS, CHUNK),
        idxx.reshape(NUM_SC, NCHUNKS, CHUNK),
        vals.reshape(NCHUNKS, CHUNK),
        zeros,
    )


# speedup vs baseline: 1.4123x; 1.4123x over previous
import functools

import jax
import jax.numpy as jnp
from jax import lax
from jax.experimental import pallas as pl
from jax.experimental.pallas import tpu as pltpu
from jax.experimental.pallas import tpu_sc as plsc

NUM_NODES = 50000
VOCAB_SIZE = 100000
EMBEDDING_DIM = 64
NNZ = 800000

NUM_SC = 2
NUM_TILES = 16
HALF = NUM_NODES // NUM_SC
ACC_ROWS = 25088
DUMMY = HALF
CHUNK = 64
CPT = 784
NCHUNKS = CPT * NUM_TILES
NNZ_PAD = NCHUNKS * CHUNK
STRIP = 8
NSTRIP = CPT // STRIP
RING = 6
ORING = 3
MROWS = 3 * STRIP
ZROWS = ACC_ROWS // NUM_TILES
WROWS = 1560

_mesh = plsc.VectorSubcoreMesh(core_axis_name="c", subcore_axis_name="s")


@functools.partial(
    pl.kernel,
    out_type=jax.ShapeDtypeStruct((NUM_NODES, EMBEDDING_DIM), jnp.float32),
    mesh=_mesh,
    scratch_types=[
        pltpu.VMEM_SHARED((ACC_ROWS, EMBEDDING_DIM), jnp.float32),
        pltpu.VMEM((MROWS, CHUNK), jnp.int32),
        pltpu.VMEM((MROWS, CHUNK), jnp.int32),
        pltpu.VMEM((MROWS, CHUNK), jnp.float32),
        pltpu.VMEM((RING, CHUNK, EMBEDDING_DIM // 2), jnp.int32),
        pltpu.VMEM((ORING, CHUNK, EMBEDDING_DIM), jnp.float32),
        pltpu.SemaphoreType.DMA,
        pltpu.SemaphoreType.DMA,
        pltpu.SemaphoreType.DMA,
    ],
    compiler_params=pltpu.CompilerParams(use_tc_tiling_on_sc=False),
)
def _sc_aggregate(table, srcx, idxx, valx, zeros, out,
                  acc, src_b, idx_b, val_b, rows_b, out_b, msem, gsem, ssem):
    c = lax.axis_index("c")
    s = lax.axis_index("s")
    node_base = c * HALF

    pltpu.sync_copy(zeros, acc.at[pl.ds(s * ZROWS, ZROWS)])
    plsc.subcore_barrier()

    chunk0 = s * CPT

    def load_strip(st, ring_slot, mode):
        g0 = chunk0 + st * STRIP
        r0 = ring_slot * STRIP
        pairs = [
            (srcx.at[pl.ds(g0, STRIP)], src_b.at[pl.ds(r0, STRIP)]),
            (idxx.at[c, pl.ds(g0, STRIP)], idx_b.at[pl.ds(r0, STRIP)]),
            (valx.at[pl.ds(g0, STRIP)], val_b.at[pl.ds(r0, STRIP)]),
        ]
        for a, b in pairs:
            if mode == "sync":
                pltpu.sync_copy(a, b)
            elif mode == "async":
                pltpu.async_copy(a, b, msem)
            else:
                pltpu.make_async_copy(a, b, msem).wait()

    def fire_gather(t):
        mrow = t % MROWS
        pltpu.async_copy(table.at[src_b.at[mrow]], rows_b.at[t % RING], gsem)

    load_strip(0, 0, "sync")
    load_strip(1, 1, "sync")
    load_strip(2, 2, "async")
    for t in range(3):
        fire_gather(t)

    @pl.loop(0, CPT)
    def step(t):
        @pl.when(t >= 3)
        def _():
            pltpu.make_async_copy(zeros.at[pl.ds(0, CHUNK)],
                                  out_b.at[(t - 3) % ORING], ssem).wait()

        st = t // STRIP

        @pl.when((t > 0) & (t % STRIP == 0))
        def _():
            @pl.when(st + 1 <= NSTRIP - 1)
            def _():
                load_strip(st + 1, (st + 1) % 3, "wait")

            @pl.when(st + 2 <= NSTRIP - 1)
            def _():
                load_strip(st + 2, (st + 2) % 3, "async")

        @pl.when(t + 3 <= CPT - 1)
        def _():
            fire_gather(t + 3)

        b = t % RING
        o = t % ORING
        mrow = t % MROWS
        pltpu.make_async_copy(table.at[src_b.at[mrow]], rows_b.at[b],
                              gsem).wait()
        himask = jnp.int32(-65536)

        @pl.loop(0, CHUNK // 16)
        def scale(i):
            vv = val_b[mrow, pl.ds(i * 16, 16)]
            for k in range(16):
                e = i * 16 + k
                sv = vv[k]
                for q in range(2):
                    wv = rows_b[b, e, pl.ds(q * 16, 16)]
                    lo = lax.bitcast_convert_type(wv << 16, jnp.float32)
                    hi = lax.bitcast_convert_type(wv & himask, jnp.float32)
                    out_b[o, e, pl.ds(q * 32, 16)] = lo * sv
                    out_b[o, e, pl.ds(q * 32 + 16, 16)] = hi * sv

        pltpu.async_copy(out_b.at[o], acc.at[idx_b.at[mrow]], ssem, add=True)

    for tt in range(CPT - 3, CPT):
        pltpu.make_async_copy(zeros.at[pl.ds(0, CHUNK)],
                              out_b.at[tt % ORING], ssem).wait()
    plsc.subcore_barrier()

    pltpu.sync_copy(acc.at[pl.ds(s * WROWS, WROWS)],
                    out.at[pl.ds(node_base + s * WROWS, WROWS)])
    rem = HALF - NUM_TILES * WROWS

    @pl.when(s == 0)
    def _():
        pltpu.sync_copy(acc.at[pl.ds(NUM_TILES * WROWS, rem)],
                        out.at[pl.ds(node_base + NUM_TILES * WROWS, rem)])


def kernel(token_embeddings, operator_values, operator_indices):
    pad = NNZ_PAD - NNZ
    dst = jnp.concatenate(
        [operator_indices[:, 0], jnp.full((pad,), NUM_NODES, jnp.int32)])
    src = jnp.concatenate(
        [operator_indices[:, 1], jnp.zeros((pad,), jnp.int32)])
    vals = jnp.concatenate(
        [operator_values, jnp.zeros((pad,), jnp.float32)])
    halves = jnp.arange(NUM_SC, dtype=jnp.int32)[:, None] * HALF
    local = dst[None, :] - halves
    idxx = jnp.where((local >= 0) & (local < HALF), local, DUMMY)
    zeros = jnp.zeros((ZROWS, EMBEDDING_DIM), jnp.float32)
    tb = token_embeddings.astype(jnp.bfloat16)
    packed = jnp.concatenate(
        [jnp.stack([tb[:, 0:16], tb[:, 16:32]], axis=-1),
         jnp.stack([tb[:, 32:48], tb[:, 48:64]], axis=-1)], axis=1)
    ti32 = lax.bitcast_convert_type(packed, jnp.int32)
    return _sc_aggregate(
        ti32,
        src.reshape(NCHUNKS, CHUNK),
        idxx.reshape(NUM_SC, NCHUNKS, CHUNK),
        vals.reshape(NCHUNKS, CHUNK),
        zeros,
    )


# speedup vs baseline: 1.4138x; 1.0011x over previous
import functools

import jax
import jax.numpy as jnp
from jax import lax
from jax.experimental import pallas as pl
from jax.experimental.pallas import tpu as pltpu
from jax.experimental.pallas import tpu_sc as plsc

NUM_NODES = 50000
VOCAB_SIZE = 100000
EMBEDDING_DIM = 64
NNZ = 800000

NUM_SC = 2
NUM_TILES = 16
HALF = NUM_NODES // NUM_SC
ACC_ROWS = 25088
DUMMY = HALF
CHUNK = 64
CPT = 784
NCHUNKS = CPT * NUM_TILES
NNZ_PAD = NCHUNKS * CHUNK
STRIP = 8
NSTRIP = CPT // STRIP
RING = 6
ORING = 3
MROWS = 3 * STRIP
ZROWS = ACC_ROWS // NUM_TILES
WROWS = 1560

_mesh = plsc.VectorSubcoreMesh(core_axis_name="c", subcore_axis_name="s")


@functools.partial(
    pl.kernel,
    out_type=jax.ShapeDtypeStruct((NUM_NODES, EMBEDDING_DIM), jnp.float32),
    mesh=_mesh,
    scratch_types=[
        pltpu.VMEM_SHARED((ACC_ROWS, EMBEDDING_DIM), jnp.float32),
        pltpu.VMEM((MROWS, CHUNK), jnp.int32),
        pltpu.VMEM((MROWS, CHUNK), jnp.int32),
        pltpu.VMEM((MROWS, CHUNK), jnp.float32),
        pltpu.VMEM((RING, CHUNK, EMBEDDING_DIM // 2), jnp.int32),
        pltpu.VMEM((ORING, CHUNK, EMBEDDING_DIM), jnp.float32),
        pltpu.SemaphoreType.DMA,
        pltpu.SemaphoreType.DMA,
        pltpu.SemaphoreType.DMA,
    ],
    compiler_params=pltpu.CompilerParams(use_tc_tiling_on_sc=False),
)
def _sc_aggregate(table, srcx, idxx, valx, zeros, out,
                  acc, src_b, idx_b, val_b, rows_b, out_b, msem, gsem, ssem):
    c = lax.axis_index("c")
    s = lax.axis_index("s")
    node_base = c * HALF

    pltpu.sync_copy(zeros, acc.at[pl.ds(s * ZROWS, ZROWS)])
    plsc.subcore_barrier()

    chunk0 = s * CPT

    def load_strip(st, ring_slot, mode):
        g0 = chunk0 + st * STRIP
        r0 = ring_slot * STRIP
        pairs = [
            (srcx.at[pl.ds(g0, STRIP)], src_b.at[pl.ds(r0, STRIP)]),
            (idxx.at[c, pl.ds(g0, STRIP)], idx_b.at[pl.ds(r0, STRIP)]),
            (valx.at[pl.ds(g0, STRIP)], val_b.at[pl.ds(r0, STRIP)]),
        ]
        for a, b in pairs:
            if mode == "sync":
                pltpu.sync_copy(a, b)
            elif mode == "async":
                pltpu.async_copy(a, b, msem)
            else:
                pltpu.make_async_copy(a, b, msem).wait()

    def fire_gather(t):
        mrow = t % MROWS
        pltpu.async_copy(table.at[src_b.at[mrow]], rows_b.at[t % RING], gsem)

    load_strip(0, 0, "sync")
    load_strip(1, 1, "sync")
    load_strip(2, 2, "async")
    for t in range(4):
        fire_gather(t)

    @pl.loop(0, CPT)
    def step(t):
        @pl.when(t >= 2)
        def _():
            pltpu.make_async_copy(zeros.at[pl.ds(0, CHUNK)],
                                  out_b.at[(t - 2) % ORING], ssem).wait()

        st = t // STRIP

        @pl.when((t > 0) & (t % STRIP == 0))
        def _():
            @pl.when(st + 1 <= NSTRIP - 1)
            def _():
                load_strip(st + 1, (st + 1) % 3, "wait")

            @pl.when(st + 2 <= NSTRIP - 1)
            def _():
                load_strip(st + 2, (st + 2) % 3, "async")

        @pl.when(t + 4 <= CPT - 1)
        def _():
            fire_gather(t + 4)

        b = t % RING
        o = t % ORING
        mrow = t % MROWS
        pltpu.make_async_copy(table.at[src_b.at[mrow]], rows_b.at[b],
                              gsem).wait()
        himask = jnp.int32(-65536)

        @pl.loop(0, CHUNK // 16)
        def scale(i):
            vv = val_b[mrow, pl.ds(i * 16, 16)]
            for k in range(16):
                e = i * 16 + k
                sv = vv[k]
                for q in range(2):
                    wv = rows_b[b, e, pl.ds(q * 16, 16)]
                    lo = lax.bitcast_convert_type(wv << 16, jnp.float32)
                    hi = lax.bitcast_convert_type(wv & himask, jnp.float32)
                    out_b[o, e, pl.ds(q * 32, 16)] = lo * sv
                    out_b[o, e, pl.ds(q * 32 + 16, 16)] = hi * sv

        pltpu.async_copy(out_b.at[o], acc.at[idx_b.at[mrow]], ssem, add=True)

    for tt in range(CPT - 2, CPT):
        pltpu.make_async_copy(zeros.at[pl.ds(0, CHUNK)],
                              out_b.at[tt % ORING], ssem).wait()
    plsc.subcore_barrier()

    pltpu.sync_copy(acc.at[pl.ds(s * WROWS, WROWS)],
                    out.at[pl.ds(node_base + s * WROWS, WROWS)])
    rem = HALF - NUM_TILES * WROWS

    @pl.when(s == 0)
    def _():
        pltpu.sync_copy(acc.at[pl.ds(NUM_TILES * WROWS, rem)],
                        out.at[pl.ds(node_base + NUM_TILES * WROWS, rem)])


def kernel(token_embeddings, operator_values, operator_indices):
    pad = NNZ_PAD - NNZ
    dst = jnp.concatenate(
        [operator_indices[:, 0], jnp.full((pad,), NUM_NODES, jnp.int32)])
    src = jnp.concatenate(
        [operator_indices[:, 1], jnp.zeros((pad,), jnp.int32)])
    vals = jnp.concatenate(
        [operator_values, jnp.zeros((pad,), jnp.float32)])
    halves = jnp.arange(NUM_SC, dtype=jnp.int32)[:, None] * HALF
    local = dst[None, :] - halves
    idxx = jnp.where((local >= 0) & (local < HALF), local, DUMMY)
    zeros = jnp.zeros((ZROWS, EMBEDDING_DIM), jnp.float32)
    tb = token_embeddings.astype(jnp.bfloat16)
    packed = jnp.concatenate(
        [jnp.stack([tb[:, 0:16], tb[:, 16:32]], axis=-1),
         jnp.stack([tb[:, 32:48], tb[:, 48:64]], axis=-1)], axis=1)
    ti32 = lax.bitcast_convert_type(packed, jnp.int32)
    return _sc_aggregate(
        ti32,
        src.reshape(NCHUNKS, CHUNK),
        idxx.reshape(NUM_SC, NCHUNKS, CHUNK),
        vals.reshape(NCHUNKS, CHUNK),
        zeros,
    )
